# baseline (speedup 1.0000x reference)
import jax
import jax.numpy as jnp
from jax.experimental import pallas as pl


def _matvec_block(x_ref, w_ref, b_ref, o_ref):
    x = x_ref[...]
    w = w_ref[...]
    d = x.shape[1]
    o_ref[...] = jnp.sum(x * w, axis=1) + (b_ref[0, 0] * d)


@jax.jit
def kernel(continuous, W_continuous, bias):
    n, d = continuous.shape
    out_dim = W_continuous.shape[1]
    w_row = W_continuous.T
    b2 = bias.reshape(1, 1)

    BN = 8192
    out = pl.pallas_call(
        _matvec_block,
        grid=(n // BN,),
        in_specs=[
            pl.BlockSpec((BN, d), lambda i: (i, 0)),
            pl.BlockSpec((1, d), lambda i: (0, 0)),
            pl.BlockSpec((1, 1), lambda i: (0, 0)),
        ],
        out_specs=pl.BlockSpec((BN,), lambda i: (i,)),
        out_shape=jax.ShapeDtypeStruct((n,), jnp.float32),
    )(continuous, w_row, b2)
    return out[:, None]


# speedup vs baseline: 1.4112x; 1.4112x over previous
import jax
import jax.numpy as jnp
from jax import lax
from jax.experimental import pallas as pl


def _matvec_block(x_ref, w_ref, b_ref, o_ref):
    x = x_ref[...]
    w = w_ref[...]
    d = x.shape[1]
    acc = lax.dot_general(
        w, x, (((1,), (1,)), ((), ())),
        preferred_element_type=jnp.float32)
    o_ref[...] = acc + b_ref[0, 0] * d


@jax.jit
def kernel(continuous, W_continuous, bias):
    n, d = continuous.shape
    out_dim = W_continuous.shape[1]
    w_row = W_continuous.T
    b2 = bias.reshape(1, 1)

    BN = 8192
    out = pl.pallas_call(
        _matvec_block,
        grid=(n // BN,),
        in_specs=[
            pl.BlockSpec((BN, d), lambda i: (i, 0)),
            pl.BlockSpec((1, d), lambda i: (0, 0)),
            pl.BlockSpec((1, 1), lambda i: (0, 0)),
        ],
        out_specs=pl.BlockSpec((1, BN), lambda i: (0, i)),
        out_shape=jax.ShapeDtypeStruct((1, n), jnp.float32),
    )(continuous, w_row, b2)
    return out.reshape(n, out_dim)
